# baseline (speedup 1.0000x reference)
import functools

import jax
import jax.numpy as jnp
from jax import lax
from jax.experimental import pallas as pl
from jax.experimental.pallas import tpu as pltpu
from jax.experimental.pallas import tpu_sc as plsc

N_NODES = 10000
N_EDGES = 320000
N_SUB = 500
N_GRAPH = 32
HID = 256
DH = 128

NS = 16
W = 128
ROWS_PER = 624
ROWS_TAIL = N_NODES - NS * ROWS_PER

NBLK = 10
BLK = N_NODES // NBLK


def _scratches(tail):
    return [
        pltpu.VMEM((W,), jnp.int32),
        pltpu.VMEM((W,), jnp.int32),
        pltpu.VMEM((W,), jnp.int32),
        pltpu.VMEM((W,), jnp.int32),
        pltpu.VMEM((W, DH), jnp.float32),
        pltpu.VMEM((W, DH), jnp.float32),
        pltpu.VMEM((tail,), jnp.int32),
        pltpu.VMEM((tail,), jnp.int32),
        pltpu.VMEM((tail, DH), jnp.float32),
        pltpu.VMEM_SHARED((N_NODES, DH), jnp.float32),
        pltpu.SemaphoreType.DMA,
        pltpu.SemaphoreType.DMA,
        pltpu.SemaphoreType.DMA,
        pltpu.SemaphoreType.DMA,
        pltpu.SemaphoreType.DMA,
        pltpu.SemaphoreType.DMA,
    ]


def _agg_subcore(h_ref, out_ref, src, dst, zer, aggs,
                 srcv0, srcv1, dstv0, dstv1, rows0, rows1,
                 srcv_t, dstv_t, rows_t,
                 issem0, issem1, idsem0, idsem1, gsem0, gsem1,
                 s, ebase, nwin, tail):
    pltpu.sync_copy(zer.at[pl.ds(s * ROWS_PER, ROWS_PER)],
                    aggs.at[pl.ds(s * ROWS_PER, ROWS_PER)])

    @pl.when(s == 0)
    def _():
        pltpu.sync_copy(zer.at[pl.ds(NS * ROWS_PER, ROWS_TAIL)],
                        aggs.at[pl.ds(NS * ROWS_PER, ROWS_TAIL)])

    plsc.subcore_barrier()

    pltpu.sync_copy(src.at[pl.ds(ebase, W)], srcv0)
    pltpu.async_copy(dst.at[pl.ds(ebase, W)], dstv0, idsem0)
    pltpu.async_copy(h_ref.at[srcv0], rows0, gsem0)
    pltpu.async_copy(src.at[pl.ds(ebase + W, W)], srcv1, issem1)
    pltpu.async_copy(dst.at[pl.ds(ebase + W, W)], dstv1, idsem1)

    def g_body(g, carry):
        o0 = ebase + (2 * g) * W
        o1 = o0 + W
        o2 = o1 + W
        o3 = o2 + W
        nl = g < nwin // 2 - 1

        pltpu.make_async_copy(src.at[pl.ds(o1, W)], srcv1, issem1).wait()
        pltpu.async_copy(h_ref.at[srcv1], rows1, gsem1)
        pltpu.make_async_copy(h_ref.at[srcv0], rows0, gsem0).wait()

        @pl.when(nl)
        def _():
            pltpu.async_copy(src.at[pl.ds(o2, W)], srcv0, issem0)

        pltpu.make_async_copy(dst.at[pl.ds(o0, W)], dstv0, idsem0).wait()
        pltpu.sync_copy(rows0, aggs.at[dstv0], add=True)

        @pl.when(nl)
        def _():
            pltpu.async_copy(dst.at[pl.ds(o2, W)], dstv0, idsem0)

        pltpu.make_async_copy(h_ref.at[srcv1], rows1, gsem1).wait()

        @pl.when(nl)
        def _():
            pltpu.async_copy(src.at[pl.ds(o3, W)], srcv1, issem1)

        pltpu.make_async_copy(dst.at[pl.ds(o1, W)], dstv1, idsem1).wait()
        pltpu.sync_copy(rows1, aggs.at[dstv1], add=True)

        @pl.when(nl)
        def _():
            pltpu.async_copy(dst.at[pl.ds(o3, W)], dstv1, idsem1)
            pltpu.make_async_copy(src.at[pl.ds(o2, W)], srcv0, issem0).wait()
            pltpu.async_copy(h_ref.at[srcv0], rows0, gsem0)

        return carry

    lax.fori_loop(0, nwin // 2, g_body, 0)

    off = ebase + nwin * W
    pltpu.sync_copy(src.at[pl.ds(off, tail)], srcv_t)
    pltpu.sync_copy(dst.at[pl.ds(off, tail)], dstv_t)
    pltpu.async_copy(h_ref.at[srcv_t], rows_t, gsem0).wait()
    pltpu.sync_copy(rows_t, aggs.at[dstv_t], add=True)
    plsc.subcore_barrier()
    pltpu.sync_copy(aggs.at[pl.ds(s * ROWS_PER, ROWS_PER)],
                    out_ref.at[pl.ds(s * ROWS_PER, ROWS_PER)])

    @pl.when(s == 0)
    def _():
        pltpu.sync_copy(aggs.at[pl.ds(NS * ROWS_PER, ROWS_TAIL)],
                        out_ref.at[pl.ds(NS * ROWS_PER, ROWS_TAIL)])


_MESH = plsc.VectorSubcoreMesh(core_axis_name="c", subcore_axis_name="s")
_OUT2 = (jax.ShapeDtypeStruct((N_NODES, DH), jnp.float32),
         jax.ShapeDtypeStruct((N_NODES, DH), jnp.float32))

_E_PER_FS = N_EDGES // NS
_NWIN_FS = _E_PER_FS // W
_TAIL_FS = _E_PER_FS - _NWIN_FS * W


@functools.partial(pl.kernel, out_type=_OUT2, mesh=_MESH,
                   scratch_types=_scratches(_TAIL_FS))
def _sc_agg_fs(h0, h1, src, dst, zer, out0, out1,
               srcv0, srcv1, dstv0, dstv1, rows0, rows1,
               srcv_t, dstv_t, rows_t, aggs,
               issem0, issem1, idsem0, idsem1, gsem0, gsem1):
    c = lax.axis_index("c")
    s = lax.axis_index("s")
    ebase = s * _E_PER_FS

    @pl.when(c == 0)
    def _():
        _agg_subcore(h0, out0, src, dst, zer, aggs,
                     srcv0, srcv1, dstv0, dstv1, rows0, rows1,
                     srcv_t, dstv_t, rows_t,
                     issem0, issem1, idsem0, idsem1, gsem0, gsem1,
                     s, ebase, _NWIN_FS, _TAIL_FS)

    @pl.when(c == 1)
    def _():
        _agg_subcore(h1, out1, src, dst, zer, aggs,
                     srcv0, srcv1, dstv0, dstv1, rows0, rows1,
                     srcv_t, dstv_t, rows_t,
                     issem0, issem1, idsem0, idsem1, gsem0, gsem1,
                     s, ebase, _NWIN_FS, _TAIL_FS)


_E_PER_ES = N_EDGES // (2 * NS)
_NWIN_ES = _E_PER_ES // W
_TAIL_ES = _E_PER_ES - _NWIN_ES * W


@functools.partial(pl.kernel, out_type=_OUT2, mesh=_MESH,
                   scratch_types=_scratches(_TAIL_ES))
def _sc_agg_es(h, src, dst, zer, out0, out1,
               srcv0, srcv1, dstv0, dstv1, rows0, rows1,
               srcv_t, dstv_t, rows_t, aggs,
               issem0, issem1, idsem0, idsem1, gsem0, gsem1):
    c = lax.axis_index("c")
    s = lax.axis_index("s")
    ebase = (c * NS + s) * _E_PER_ES

    @pl.when(c == 0)
    def _():
        _agg_subcore(h, out0, src, dst, zer, aggs,
                     srcv0, srcv1, dstv0, dstv1, rows0, rows1,
                     srcv_t, dstv_t, rows_t,
                     issem0, issem1, idsem0, idsem1, gsem0, gsem1,
                     s, ebase, _NWIN_ES, _TAIL_ES)

    @pl.when(c == 1)
    def _():
        _agg_subcore(h, out1, src, dst, zer, aggs,
                     srcv0, srcv1, dstv0, dstv1, rows0, rows1,
                     srcv_t, dstv_t, rows_t,
                     issem0, issem1, idsem0, idsem1, gsem0, gsem1,
                     s, ebase, _NWIN_ES, _TAIL_ES)


def _mlp0_body(h, a0, a1, sc, w1, b1, w2, b2, y0, y1):
    z = sc[0, 0] * h[...] + a0[...] + a1[...]
    t = jnp.maximum(
        jnp.dot(z, w1[...], preferred_element_type=jnp.float32) + b1[...], 0.0)
    y = jnp.dot(t, w2[...], preferred_element_type=jnp.float32) + b2[...]
    y0[...] = y[:, :HID // 2]
    y1[...] = y[:, HID // 2:]


_mlp0 = pl.pallas_call(
    _mlp0_body,
    grid=(NBLK,),
    in_specs=[
        pl.BlockSpec((BLK, DH), lambda i: (i, 0)),
        pl.BlockSpec((BLK, DH), lambda i: (i, 0)),
        pl.BlockSpec((BLK, DH), lambda i: (i, 0)),
        pl.BlockSpec((1, 1), lambda i: (0, 0)),
        pl.BlockSpec((DH, HID), lambda i: (0, 0)),
        pl.BlockSpec((HID,), lambda i: (0,)),
        pl.BlockSpec((HID, HID), lambda i: (0, 0)),
        pl.BlockSpec((HID,), lambda i: (0,)),
    ],
    out_specs=[
        pl.BlockSpec((BLK, HID // 2), lambda i: (i, 0)),
        pl.BlockSpec((BLK, HID // 2), lambda i: (i, 0)),
    ],
    out_shape=[
        jax.ShapeDtypeStruct((N_NODES, HID // 2), jnp.float32),
        jax.ShapeDtypeStruct((N_NODES, HID // 2), jnp.float32),
    ],
)


def _mlp2_body(h0, h1, a0, a1, sc, w1, b1, w2, b2, y0, y1):
    z = jnp.concatenate(
        [sc[0, 0] * h0[...] + a0[...], sc[0, 0] * h1[...] + a1[...]], axis=1)
    t = jnp.maximum(
        jnp.dot(z, w1[...], preferred_element_type=jnp.float32) + b1[...], 0.0)
    y = jnp.dot(t, w2[...], preferred_element_type=jnp.float32) + b2[...]
    y0[...] = y[:, :HID // 2]
    y1[...] = y[:, HID // 2:]


_mlp2 = pl.pallas_call(
    _mlp2_body,
    grid=(NBLK,),
    in_specs=[
        pl.BlockSpec((BLK, HID // 2), lambda i: (i, 0)),
        pl.BlockSpec((BLK, HID // 2), lambda i: (i, 0)),
        pl.BlockSpec((BLK, HID // 2), lambda i: (i, 0)),
        pl.BlockSpec((BLK, HID // 2), lambda i: (i, 0)),
        pl.BlockSpec((1, 1), lambda i: (0, 0)),
        pl.BlockSpec((HID, HID), lambda i: (0, 0)),
        pl.BlockSpec((HID,), lambda i: (0,)),
        pl.BlockSpec((HID, HID), lambda i: (0, 0)),
        pl.BlockSpec((HID,), lambda i: (0,)),
    ],
    out_specs=[
        pl.BlockSpec((BLK, HID // 2), lambda i: (i, 0)),
        pl.BlockSpec((BLK, HID // 2), lambda i: (i, 0)),
    ],
    out_shape=[
        jax.ShapeDtypeStruct((N_NODES, HID // 2), jnp.float32),
        jax.ShapeDtypeStruct((N_NODES, HID // 2), jnp.float32),
    ],
)


def _head_body(y00, y01, y10, y11, y20, y21, bat, s2g,
               l1w, l1b, l2w, l2b, out, sub_acc):
    i = pl.program_id(0)
    xb = jnp.concatenate(
        [y00[...], y01[...], y10[...], y11[...], y20[...], y21[...]], axis=1)
    m = (lax.broadcasted_iota(jnp.int32, (N_SUB, BLK), 0)
         == bat[0]).astype(jnp.float32)
    part = jnp.dot(m, xb, preferred_element_type=jnp.float32)

    @pl.when(i == 0)
    def _():
        sub_acc[...] = part

    @pl.when(i > 0)
    def _():
        sub_acc[...] += part

    @pl.when(i == NBLK - 1)
    def _():
        mg = (lax.broadcasted_iota(jnp.int32, (N_GRAPH, N_SUB), 0)
              == s2g[0]).astype(jnp.float32)
        g = jnp.dot(mg, sub_acc[...], preferred_element_type=jnp.float32)
        t = jnp.maximum(
            jnp.dot(g, l1w[...], preferred_element_type=jnp.float32) + l1b[...],
            0.0)
        o = jnp.dot(t, l2w[...], preferred_element_type=jnp.float32) + l2b[...]
        mx = jnp.max(o, axis=1, keepdims=True)
        lse = jnp.log(jnp.sum(jnp.exp(o - mx), axis=1, keepdims=True)) + mx
        out[...] = o - lse


_head = pl.pallas_call(
    _head_body,
    grid=(NBLK,),
    in_specs=[pl.BlockSpec((BLK, HID // 2), lambda i: (i, 0))] * 6 + [
        pl.BlockSpec((1, 1, BLK), lambda i: (i, 0, 0)),
        pl.BlockSpec((1, 1, N_SUB), lambda i: (0, 0, 0)),
        pl.BlockSpec((3 * HID, HID), lambda i: (0, 0)),
        pl.BlockSpec((HID,), lambda i: (0,)),
        pl.BlockSpec((HID, HID), lambda i: (0, 0)),
        pl.BlockSpec((HID,), lambda i: (0,)),
    ],
    out_specs=pl.BlockSpec((N_GRAPH, HID), lambda i: (0, 0)),
    out_shape=jax.ShapeDtypeStruct((N_GRAPH, HID), jnp.float32),
    scratch_shapes=[pltpu.VMEM((N_SUB, 3 * HID), jnp.float32)],
)


def kernel(x, edge_index, batch, subgraph_to_graph,
           W1_0, b1_0, W2_0, b2_0, eps_0,
           W1_1, b1_1, W2_1, b2_1, eps_1,
           W1_2, b1_2, W2_2, b2_2, eps_2,
           lin1_W, lin1_b, lin2_W, lin2_b):
    src = edge_index[0]
    dst = edge_index[1]
    zer = jnp.zeros((N_NODES, DH), jnp.float32)
    bat3 = batch.astype(jnp.int32).reshape(NBLK, 1, BLK)
    s2g3 = subgraph_to_graph.astype(jnp.int32).reshape(1, 1, N_SUB)

    a0, a1 = _sc_agg_es(x, src, dst, zer)
    y00, y01 = _mlp0(x, a0, a1, (1.0 + eps_0).reshape(1, 1),
                     W1_0, b1_0, W2_0, b2_0)

    a0, a1 = _sc_agg_fs(y00, y01, src, dst, zer)
    y10, y11 = _mlp2(y00, y01, a0, a1, (1.0 + eps_1).reshape(1, 1),
                     W1_1, b1_1, W2_1, b2_1)

    a0, a1 = _sc_agg_fs(y10, y11, src, dst, zer)
    y20, y21 = _mlp2(y10, y11, a0, a1, (1.0 + eps_2).reshape(1, 1),
                     W1_2, b1_2, W2_2, b2_2)

    return _head(y00, y01, y10, y11, y20, y21, bat3, s2g3,
                 lin1_W, lin1_b, lin2_W, lin2_b)


# speedup vs baseline: 1.0039x; 1.0039x over previous
import functools

import jax
import jax.numpy as jnp
from jax import lax
from jax.experimental import pallas as pl
from jax.experimental.pallas import tpu as pltpu
from jax.experimental.pallas import tpu_sc as plsc

N_NODES = 10000
N_EDGES = 320000
N_SUB = 500
N_GRAPH = 32
HID = 256
DH = 128

NS = 16
W = 128
ROWS_PER = 624
ROWS_TAIL = N_NODES - NS * ROWS_PER

NBLK = 10
BLK = N_NODES // NBLK


def _scratches(tail):
    return [
        pltpu.VMEM((W,), jnp.int32),
        pltpu.VMEM((W,), jnp.int32),
        pltpu.VMEM((W,), jnp.int32),
        pltpu.VMEM((W,), jnp.int32),
        pltpu.VMEM((W, DH), jnp.float32),
        pltpu.VMEM((W, DH), jnp.float32),
        pltpu.VMEM((tail,), jnp.int32),
        pltpu.VMEM((tail,), jnp.int32),
        pltpu.VMEM((tail, DH), jnp.float32),
        pltpu.VMEM_SHARED((N_NODES, DH), jnp.float32),
        pltpu.SemaphoreType.DMA,
        pltpu.SemaphoreType.DMA,
        pltpu.SemaphoreType.DMA,
        pltpu.SemaphoreType.DMA,
        pltpu.SemaphoreType.DMA,
        pltpu.SemaphoreType.DMA,
    ]


def _agg_subcore(h_ref, out_ref, src, dst, zer, aggs,
                 srcv0, srcv1, dstv0, dstv1, rows0, rows1,
                 srcv_t, dstv_t, rows_t,
                 issem0, issem1, idsem0, idsem1, gsem0, gsem1,
                 s, ebase, nwin, tail):
    pltpu.sync_copy(zer.at[pl.ds(s * ROWS_PER, ROWS_PER)],
                    aggs.at[pl.ds(s * ROWS_PER, ROWS_PER)])

    @pl.when(s == 0)
    def _():
        pltpu.sync_copy(zer.at[pl.ds(NS * ROWS_PER, ROWS_TAIL)],
                        aggs.at[pl.ds(NS * ROWS_PER, ROWS_TAIL)])

    plsc.subcore_barrier()

    pltpu.sync_copy(src.at[pl.ds(ebase, W)], srcv0)
    pltpu.async_copy(dst.at[pl.ds(ebase, W)], dstv0, idsem0)
    pltpu.async_copy(h_ref.at[srcv0], rows0, gsem0)
    pltpu.async_copy(src.at[pl.ds(ebase + W, W)], srcv1, issem1)
    pltpu.async_copy(dst.at[pl.ds(ebase + W, W)], dstv1, idsem1)

    def g_body(g, carry):
        o0 = ebase + (2 * g) * W
        o1 = o0 + W
        o2 = o1 + W
        o3 = o2 + W
        nl = g < nwin // 2 - 1

        pltpu.make_async_copy(src.at[pl.ds(o1, W)], srcv1, issem1).wait()
        pltpu.async_copy(h_ref.at[srcv1], rows1, gsem1)
        pltpu.make_async_copy(h_ref.at[srcv0], rows0, gsem0).wait()

        @pl.when(nl)
        def _():
            pltpu.async_copy(src.at[pl.ds(o2, W)], srcv0, issem0)

        pltpu.make_async_copy(dst.at[pl.ds(o0, W)], dstv0, idsem0).wait()
        pltpu.sync_copy(rows0, aggs.at[dstv0], add=True)

        @pl.when(nl)
        def _():
            pltpu.async_copy(dst.at[pl.ds(o2, W)], dstv0, idsem0)

        pltpu.make_async_copy(h_ref.at[srcv1], rows1, gsem1).wait()

        @pl.when(nl)
        def _():
            pltpu.async_copy(src.at[pl.ds(o3, W)], srcv1, issem1)

        pltpu.make_async_copy(dst.at[pl.ds(o1, W)], dstv1, idsem1).wait()
        pltpu.sync_copy(rows1, aggs.at[dstv1], add=True)

        @pl.when(nl)
        def _():
            pltpu.async_copy(dst.at[pl.ds(o3, W)], dstv1, idsem1)
            pltpu.make_async_copy(src.at[pl.ds(o2, W)], srcv0, issem0).wait()
            pltpu.async_copy(h_ref.at[srcv0], rows0, gsem0)

        return carry

    lax.fori_loop(0, nwin // 2, g_body, 0)

    off = ebase + nwin * W
    pltpu.sync_copy(src.at[pl.ds(off, tail)], srcv_t)
    pltpu.sync_copy(dst.at[pl.ds(off, tail)], dstv_t)
    pltpu.async_copy(h_ref.at[srcv_t], rows_t, gsem0).wait()
    pltpu.sync_copy(rows_t, aggs.at[dstv_t], add=True)
    plsc.subcore_barrier()
    pltpu.sync_copy(aggs.at[pl.ds(s * ROWS_PER, ROWS_PER)],
                    out_ref.at[pl.ds(s * ROWS_PER, ROWS_PER)])

    @pl.when(s == 0)
    def _():
        pltpu.sync_copy(aggs.at[pl.ds(NS * ROWS_PER, ROWS_TAIL)],
                        out_ref.at[pl.ds(NS * ROWS_PER, ROWS_TAIL)])


_MESH = plsc.VectorSubcoreMesh(core_axis_name="c", subcore_axis_name="s")
_OUT2 = (jax.ShapeDtypeStruct((N_NODES, DH), jnp.float32),
         jax.ShapeDtypeStruct((N_NODES, DH), jnp.float32))

_E_PER_FS = N_EDGES // NS
_NWIN_FS = _E_PER_FS // W
_TAIL_FS = _E_PER_FS - _NWIN_FS * W


@functools.partial(pl.kernel, out_type=_OUT2, mesh=_MESH,
                   scratch_types=_scratches(_TAIL_FS))
def _sc_agg_fs(h0, h1, src, dst, zer, out0, out1,
               srcv0, srcv1, dstv0, dstv1, rows0, rows1,
               srcv_t, dstv_t, rows_t, aggs,
               issem0, issem1, idsem0, idsem1, gsem0, gsem1):
    c = lax.axis_index("c")
    s = lax.axis_index("s")
    ebase = s * _E_PER_FS

    @pl.when(c == 0)
    def _():
        _agg_subcore(h0, out0, src, dst, zer, aggs,
                     srcv0, srcv1, dstv0, dstv1, rows0, rows1,
                     srcv_t, dstv_t, rows_t,
                     issem0, issem1, idsem0, idsem1, gsem0, gsem1,
                     s, ebase, _NWIN_FS, _TAIL_FS)

    @pl.when(c == 1)
    def _():
        _agg_subcore(h1, out1, src, dst, zer, aggs,
                     srcv0, srcv1, dstv0, dstv1, rows0, rows1,
                     srcv_t, dstv_t, rows_t,
                     issem0, issem1, idsem0, idsem1, gsem0, gsem1,
                     s, ebase, _NWIN_FS, _TAIL_FS)


_E_PER_ES = N_EDGES // (2 * NS)
_NWIN_ES = _E_PER_ES // W
_TAIL_ES = _E_PER_ES - _NWIN_ES * W


@functools.partial(pl.kernel, out_type=_OUT2, mesh=_MESH,
                   scratch_types=_scratches(_TAIL_ES))
def _sc_agg_es(h, src, dst, zer, out0, out1,
               srcv0, srcv1, dstv0, dstv1, rows0, rows1,
               srcv_t, dstv_t, rows_t, aggs,
               issem0, issem1, idsem0, idsem1, gsem0, gsem1):
    c = lax.axis_index("c")
    s = lax.axis_index("s")
    ebase = (c * NS + s) * _E_PER_ES

    @pl.when(c == 0)
    def _():
        _agg_subcore(h, out0, src, dst, zer, aggs,
                     srcv0, srcv1, dstv0, dstv1, rows0, rows1,
                     srcv_t, dstv_t, rows_t,
                     issem0, issem1, idsem0, idsem1, gsem0, gsem1,
                     s, ebase, _NWIN_ES, _TAIL_ES)

    @pl.when(c == 1)
    def _():
        _agg_subcore(h, out1, src, dst, zer, aggs,
                     srcv0, srcv1, dstv0, dstv1, rows0, rows1,
                     srcv_t, dstv_t, rows_t,
                     issem0, issem1, idsem0, idsem1, gsem0, gsem1,
                     s, ebase, _NWIN_ES, _TAIL_ES)


def _mlp0_body(h, a0, a1, sc, w1, b1, w2, b2, y0, y1):
    z = sc[0, 0] * h[...] + a0[...] + a1[...]
    t = jnp.maximum(
        jnp.dot(z.astype(jnp.bfloat16), w1[...],
                preferred_element_type=jnp.float32) + b1[...], 0.0)
    y = jnp.dot(t.astype(jnp.bfloat16), w2[...],
                preferred_element_type=jnp.float32) + b2[...]
    y0[...] = y[:, :HID // 2]
    y1[...] = y[:, HID // 2:]


_mlp0 = pl.pallas_call(
    _mlp0_body,
    grid=(NBLK,),
    in_specs=[
        pl.BlockSpec((BLK, DH), lambda i: (i, 0)),
        pl.BlockSpec((BLK, DH), lambda i: (i, 0)),
        pl.BlockSpec((BLK, DH), lambda i: (i, 0)),
        pl.BlockSpec((1, 1), lambda i: (0, 0)),
        pl.BlockSpec((DH, HID), lambda i: (0, 0)),
        pl.BlockSpec((HID,), lambda i: (0,)),
        pl.BlockSpec((HID, HID), lambda i: (0, 0)),
        pl.BlockSpec((HID,), lambda i: (0,)),
    ],
    out_specs=[
        pl.BlockSpec((BLK, HID // 2), lambda i: (i, 0)),
        pl.BlockSpec((BLK, HID // 2), lambda i: (i, 0)),
    ],
    out_shape=[
        jax.ShapeDtypeStruct((N_NODES, HID // 2), jnp.float32),
        jax.ShapeDtypeStruct((N_NODES, HID // 2), jnp.float32),
    ],
)


def _mlp2_body(h0, h1, a0, a1, sc, w1, b1, w2, b2, y0, y1):
    z = jnp.concatenate(
        [sc[0, 0] * h0[...] + a0[...], sc[0, 0] * h1[...] + a1[...]], axis=1)
    t = jnp.maximum(
        jnp.dot(z.astype(jnp.bfloat16), w1[...],
                preferred_element_type=jnp.float32) + b1[...], 0.0)
    y = jnp.dot(t.astype(jnp.bfloat16), w2[...],
                preferred_element_type=jnp.float32) + b2[...]
    y0[...] = y[:, :HID // 2]
    y1[...] = y[:, HID // 2:]


_mlp2 = pl.pallas_call(
    _mlp2_body,
    grid=(NBLK,),
    in_specs=[
        pl.BlockSpec((BLK, HID // 2), lambda i: (i, 0)),
        pl.BlockSpec((BLK, HID // 2), lambda i: (i, 0)),
        pl.BlockSpec((BLK, HID // 2), lambda i: (i, 0)),
        pl.BlockSpec((BLK, HID // 2), lambda i: (i, 0)),
        pl.BlockSpec((1, 1), lambda i: (0, 0)),
        pl.BlockSpec((HID, HID), lambda i: (0, 0)),
        pl.BlockSpec((HID,), lambda i: (0,)),
        pl.BlockSpec((HID, HID), lambda i: (0, 0)),
        pl.BlockSpec((HID,), lambda i: (0,)),
    ],
    out_specs=[
        pl.BlockSpec((BLK, HID // 2), lambda i: (i, 0)),
        pl.BlockSpec((BLK, HID // 2), lambda i: (i, 0)),
    ],
    out_shape=[
        jax.ShapeDtypeStruct((N_NODES, HID // 2), jnp.float32),
        jax.ShapeDtypeStruct((N_NODES, HID // 2), jnp.float32),
    ],
)


def _pool_body(y0, y1, bat, pool, acc):
    i = pl.program_id(0)
    xb = jnp.concatenate([y0[...], y1[...]], axis=1).astype(jnp.bfloat16)
    m = (lax.broadcasted_iota(jnp.int32, (N_SUB, BLK), 0)
         == bat[0]).astype(jnp.bfloat16)
    part = jnp.dot(m, xb, preferred_element_type=jnp.float32)

    @pl.when(i == 0)
    def _():
        acc[...] = part

    @pl.when(i > 0)
    def _():
        acc[...] += part

    @pl.when(i == NBLK - 1)
    def _():
        pool[...] = acc[...]


_pool = pl.pallas_call(
    _pool_body,
    grid=(NBLK,),
    in_specs=[
        pl.BlockSpec((BLK, HID // 2), lambda i: (i, 0)),
        pl.BlockSpec((BLK, HID // 2), lambda i: (i, 0)),
        pl.BlockSpec((1, 1, BLK), lambda i: (i, 0, 0)),
    ],
    out_specs=pl.BlockSpec((N_SUB, HID), lambda i: (0, 0)),
    out_shape=jax.ShapeDtypeStruct((N_SUB, HID), jnp.float32),
    scratch_shapes=[pltpu.VMEM((N_SUB, HID), jnp.float32)],
)


def _head_body(p0, p1, p2, s2g, l1w, l1b, l2w, l2b, out):
    sub = jnp.concatenate([p0[...], p1[...], p2[...]], axis=1)
    mg = (lax.broadcasted_iota(jnp.int32, (N_GRAPH, N_SUB), 0)
          == s2g[0, 0]).astype(jnp.float32)
    g = jnp.dot(mg, sub, preferred_element_type=jnp.float32)
    t = jnp.maximum(
        jnp.dot(g, l1w[...], preferred_element_type=jnp.float32) + l1b[...],
        0.0)
    o = jnp.dot(t, l2w[...], preferred_element_type=jnp.float32) + l2b[...]
    mx = jnp.max(o, axis=1, keepdims=True)
    lse = jnp.log(jnp.sum(jnp.exp(o - mx), axis=1, keepdims=True)) + mx
    out[...] = o - lse


_head = pl.pallas_call(
    _head_body,
    out_shape=jax.ShapeDtypeStruct((N_GRAPH, HID), jnp.float32),
)


def kernel(x, edge_index, batch, subgraph_to_graph,
           W1_0, b1_0, W2_0, b2_0, eps_0,
           W1_1, b1_1, W2_1, b2_1, eps_1,
           W1_2, b1_2, W2_2, b2_2, eps_2,
           lin1_W, lin1_b, lin2_W, lin2_b):
    src = edge_index[0]
    dst = edge_index[1]
    zer = jnp.zeros((N_NODES, DH), jnp.float32)
    bat3 = batch.astype(jnp.int32).reshape(NBLK, 1, BLK)
    s2g3 = subgraph_to_graph.astype(jnp.int32).reshape(1, 1, N_SUB)

    a0, a1 = _sc_agg_es(x, src, dst, zer)
    y00, y01 = _mlp0(x, a0, a1, (1.0 + eps_0).reshape(1, 1),
                     W1_0.astype(jnp.bfloat16), b1_0,
                     W2_0.astype(jnp.bfloat16), b2_0)
    p0 = _pool(y00, y01, bat3)

    a0, a1 = _sc_agg_fs(y00, y01, src, dst, zer)
    y10, y11 = _mlp2(y00, y01, a0, a1, (1.0 + eps_1).reshape(1, 1),
                     W1_1.astype(jnp.bfloat16), b1_1,
                     W2_1.astype(jnp.bfloat16), b2_1)
    p1 = _pool(y10, y11, bat3)

    a0, a1 = _sc_agg_fs(y10, y11, src, dst, zer)
    y20, y21 = _mlp2(y10, y11, a0, a1, (1.0 + eps_2).reshape(1, 1),
                     W1_2.astype(jnp.bfloat16), b1_2,
                     W2_2.astype(jnp.bfloat16), b2_2)
    p2 = _pool(y20, y21, bat3)

    return _head(p0, p1, p2, s2g3, lin1_W, lin1_b, lin2_W, lin2_b)


# speedup vs baseline: 1.0186x; 1.0146x over previous
import functools

import jax
import jax.numpy as jnp
from jax import lax
from jax.experimental import pallas as pl
from jax.experimental.pallas import tpu as pltpu
from jax.experimental.pallas import tpu_sc as plsc

N_NODES = 10000
N_EDGES = 320000
N_SUB = 500
N_GRAPH = 32
HID = 256
DH = 128

NS = 16
W = 128
ROWS_PER = 624
ROWS_TAIL = N_NODES - NS * ROWS_PER

NBLK = 10
BLK = N_NODES // NBLK


def _scratches(tail):
    return [
        pltpu.VMEM((W,), jnp.int32),
        pltpu.VMEM((W,), jnp.int32),
        pltpu.VMEM((W,), jnp.int32),
        pltpu.VMEM((W,), jnp.int32),
        pltpu.VMEM((W, DH), jnp.float32),
        pltpu.VMEM((W, DH), jnp.float32),
        pltpu.VMEM((tail,), jnp.int32),
        pltpu.VMEM((tail,), jnp.int32),
        pltpu.VMEM((tail, DH), jnp.float32),
        pltpu.VMEM_SHARED((N_NODES, DH), jnp.float32),
        pltpu.SemaphoreType.DMA,
        pltpu.SemaphoreType.DMA,
        pltpu.SemaphoreType.DMA,
        pltpu.SemaphoreType.DMA,
        pltpu.SemaphoreType.DMA,
        pltpu.SemaphoreType.DMA,
    ]


def _agg_subcore(h_ref, out_ref, src, dst, zer, aggs,
                 srcv0, srcv1, dstv0, dstv1, rows0, rows1,
                 srcv_t, dstv_t, rows_t,
                 issem0, issem1, idsem0, idsem1, gsem0, gsem1,
                 s, ebase, nwin, tail):
    pltpu.sync_copy(zer.at[pl.ds(s * ROWS_PER, ROWS_PER)],
                    aggs.at[pl.ds(s * ROWS_PER, ROWS_PER)])

    @pl.when(s == 0)
    def _():
        pltpu.sync_copy(zer.at[pl.ds(NS * ROWS_PER, ROWS_TAIL)],
                        aggs.at[pl.ds(NS * ROWS_PER, ROWS_TAIL)])

    plsc.subcore_barrier()

    pltpu.sync_copy(src.at[pl.ds(ebase, W)], srcv0)
    pltpu.async_copy(dst.at[pl.ds(ebase, W)], dstv0, idsem0)
    pltpu.async_copy(h_ref.at[srcv0], rows0, gsem0)
    pltpu.async_copy(src.at[pl.ds(ebase + W, W)], srcv1, issem1)
    pltpu.async_copy(dst.at[pl.ds(ebase + W, W)], dstv1, idsem1)

    def g_body(g, carry):
        o0 = ebase + (2 * g) * W
        o1 = o0 + W
        o2 = o1 + W
        o3 = o2 + W
        nl = g < nwin // 2 - 1

        pltpu.make_async_copy(src.at[pl.ds(o1, W)], srcv1, issem1).wait()
        pltpu.async_copy(h_ref.at[srcv1], rows1, gsem1)
        pltpu.make_async_copy(h_ref.at[srcv0], rows0, gsem0).wait()

        @pl.when(nl)
        def _():
            pltpu.async_copy(src.at[pl.ds(o2, W)], srcv0, issem0)

        pltpu.make_async_copy(dst.at[pl.ds(o0, W)], dstv0, idsem0).wait()
        pltpu.sync_copy(rows0, aggs.at[dstv0], add=True)

        @pl.when(nl)
        def _():
            pltpu.async_copy(dst.at[pl.ds(o2, W)], dstv0, idsem0)

        pltpu.make_async_copy(h_ref.at[srcv1], rows1, gsem1).wait()

        @pl.when(nl)
        def _():
            pltpu.async_copy(src.at[pl.ds(o3, W)], srcv1, issem1)

        pltpu.make_async_copy(dst.at[pl.ds(o1, W)], dstv1, idsem1).wait()
        pltpu.sync_copy(rows1, aggs.at[dstv1], add=True)

        @pl.when(nl)
        def _():
            pltpu.async_copy(dst.at[pl.ds(o3, W)], dstv1, idsem1)
            pltpu.make_async_copy(src.at[pl.ds(o2, W)], srcv0, issem0).wait()
            pltpu.async_copy(h_ref.at[srcv0], rows0, gsem0)

        return carry

    lax.fori_loop(0, nwin // 2, g_body, 0)

    off = ebase + nwin * W
    pltpu.sync_copy(src.at[pl.ds(off, tail)], srcv_t)
    pltpu.sync_copy(dst.at[pl.ds(off, tail)], dstv_t)
    pltpu.async_copy(h_ref.at[srcv_t], rows_t, gsem0).wait()
    pltpu.sync_copy(rows_t, aggs.at[dstv_t], add=True)
    plsc.subcore_barrier()
    pltpu.sync_copy(aggs.at[pl.ds(s * ROWS_PER, ROWS_PER)],
                    out_ref.at[pl.ds(s * ROWS_PER, ROWS_PER)])

    @pl.when(s == 0)
    def _():
        pltpu.sync_copy(aggs.at[pl.ds(NS * ROWS_PER, ROWS_TAIL)],
                        out_ref.at[pl.ds(NS * ROWS_PER, ROWS_TAIL)])


_MESH = plsc.VectorSubcoreMesh(core_axis_name="c", subcore_axis_name="s")
_OUT2 = (jax.ShapeDtypeStruct((N_NODES, DH), jnp.float32),
         jax.ShapeDtypeStruct((N_NODES, DH), jnp.float32))

_E_PER_FS = N_EDGES // NS
_NWIN_FS = _E_PER_FS // W
_TAIL_FS = _E_PER_FS - _NWIN_FS * W


@functools.partial(pl.kernel, out_type=_OUT2, mesh=_MESH,
                   scratch_types=_scratches(_TAIL_FS))
def _sc_agg_fs(h0, h1, src, dst, zer, out0, out1,
               srcv0, srcv1, dstv0, dstv1, rows0, rows1,
               srcv_t, dstv_t, rows_t, aggs,
               issem0, issem1, idsem0, idsem1, gsem0, gsem1):
    c = lax.axis_index("c")
    s = lax.axis_index("s")
    ebase = s * _E_PER_FS

    @pl.when(c == 0)
    def _():
        _agg_subcore(h0, out0, src, dst, zer, aggs,
                     srcv0, srcv1, dstv0, dstv1, rows0, rows1,
                     srcv_t, dstv_t, rows_t,
                     issem0, issem1, idsem0, idsem1, gsem0, gsem1,
                     s, ebase, _NWIN_FS, _TAIL_FS)

    @pl.when(c == 1)
    def _():
        _agg_subcore(h1, out1, src, dst, zer, aggs,
                     srcv0, srcv1, dstv0, dstv1, rows0, rows1,
                     srcv_t, dstv_t, rows_t,
                     issem0, issem1, idsem0, idsem1, gsem0, gsem1,
                     s, ebase, _NWIN_FS, _TAIL_FS)


_E_PER_ES = N_EDGES // (2 * NS)
_NWIN_ES = _E_PER_ES // W
_TAIL_ES = _E_PER_ES - _NWIN_ES * W


@functools.partial(pl.kernel, out_type=_OUT2, mesh=_MESH,
                   scratch_types=_scratches(_TAIL_ES))
def _sc_agg_es(h, src, dst, zer, out0, out1,
               srcv0, srcv1, dstv0, dstv1, rows0, rows1,
               srcv_t, dstv_t, rows_t, aggs,
               issem0, issem1, idsem0, idsem1, gsem0, gsem1):
    c = lax.axis_index("c")
    s = lax.axis_index("s")
    ebase = (c * NS + s) * _E_PER_ES

    @pl.when(c == 0)
    def _():
        _agg_subcore(h, out0, src, dst, zer, aggs,
                     srcv0, srcv1, dstv0, dstv1, rows0, rows1,
                     srcv_t, dstv_t, rows_t,
                     issem0, issem1, idsem0, idsem1, gsem0, gsem1,
                     s, ebase, _NWIN_ES, _TAIL_ES)

    @pl.when(c == 1)
    def _():
        _agg_subcore(h, out1, src, dst, zer, aggs,
                     srcv0, srcv1, dstv0, dstv1, rows0, rows1,
                     srcv_t, dstv_t, rows_t,
                     issem0, issem1, idsem0, idsem1, gsem0, gsem1,
                     s, ebase, _NWIN_ES, _TAIL_ES)


def _mlp0_body(h, a0, a1, sc, w1, b1, w2, b2, y0, y1):
    z = sc[0, 0] * h[...] + a0[...] + a1[...]
    t = jnp.maximum(
        jnp.dot(z.astype(jnp.bfloat16), w1[...],
                preferred_element_type=jnp.float32) + b1[...], 0.0)
    y = jnp.dot(t.astype(jnp.bfloat16), w2[...],
                preferred_element_type=jnp.float32) + b2[...]
    y0[...] = y[:, :HID // 2]
    y1[...] = y[:, HID // 2:]


_mlp0 = pl.pallas_call(
    _mlp0_body,
    grid=(NBLK,),
    in_specs=[
        pl.BlockSpec((BLK, DH), lambda i: (i, 0)),
        pl.BlockSpec((BLK, DH), lambda i: (i, 0)),
        pl.BlockSpec((BLK, DH), lambda i: (i, 0)),
        pl.BlockSpec((1, 1), lambda i: (0, 0)),
        pl.BlockSpec((DH, HID), lambda i: (0, 0)),
        pl.BlockSpec((HID,), lambda i: (0,)),
        pl.BlockSpec((HID, HID), lambda i: (0, 0)),
        pl.BlockSpec((HID,), lambda i: (0,)),
    ],
    out_specs=[
        pl.BlockSpec((BLK, HID // 2), lambda i: (i, 0)),
        pl.BlockSpec((BLK, HID // 2), lambda i: (i, 0)),
    ],
    out_shape=[
        jax.ShapeDtypeStruct((N_NODES, HID // 2), jnp.float32),
        jax.ShapeDtypeStruct((N_NODES, HID // 2), jnp.float32),
    ],
)


def _mlp2_body(h0, h1, a0, a1, sc, w1, b1, w2, b2, y0, y1):
    z = jnp.concatenate(
        [sc[0, 0] * h0[...] + a0[...], sc[0, 0] * h1[...] + a1[...]], axis=1)
    t = jnp.maximum(
        jnp.dot(z.astype(jnp.bfloat16), w1[...],
                preferred_element_type=jnp.float32) + b1[...], 0.0)
    y = jnp.dot(t.astype(jnp.bfloat16), w2[...],
                preferred_element_type=jnp.float32) + b2[...]
    y0[...] = y[:, :HID // 2]
    y1[...] = y[:, HID // 2:]


_mlp2 = pl.pallas_call(
    _mlp2_body,
    grid=(NBLK,),
    in_specs=[
        pl.BlockSpec((BLK, HID // 2), lambda i: (i, 0)),
        pl.BlockSpec((BLK, HID // 2), lambda i: (i, 0)),
        pl.BlockSpec((BLK, HID // 2), lambda i: (i, 0)),
        pl.BlockSpec((BLK, HID // 2), lambda i: (i, 0)),
        pl.BlockSpec((1, 1), lambda i: (0, 0)),
        pl.BlockSpec((HID, HID), lambda i: (0, 0)),
        pl.BlockSpec((HID,), lambda i: (0,)),
        pl.BlockSpec((HID, HID), lambda i: (0, 0)),
        pl.BlockSpec((HID,), lambda i: (0,)),
    ],
    out_specs=[
        pl.BlockSpec((BLK, HID // 2), lambda i: (i, 0)),
        pl.BlockSpec((BLK, HID // 2), lambda i: (i, 0)),
    ],
    out_shape=[
        jax.ShapeDtypeStruct((N_NODES, HID // 2), jnp.float32),
        jax.ShapeDtypeStruct((N_NODES, HID // 2), jnp.float32),
    ],
)


def _pool_body(y0, y1, bat, pool, acc):
    i = pl.program_id(0)
    xb = jnp.concatenate([y0[...], y1[...]], axis=1).astype(jnp.bfloat16)
    m = (lax.broadcasted_iota(jnp.int32, (N_SUB, BLK), 0)
         == bat[0]).astype(jnp.bfloat16)
    part = jnp.dot(m, xb, preferred_element_type=jnp.float32)

    @pl.when(i == 0)
    def _():
        acc[...] = part

    @pl.when(i > 0)
    def _():
        acc[...] += part

    @pl.when(i == NBLK - 1)
    def _():
        pool[...] = acc[...]


_pool = pl.pallas_call(
    _pool_body,
    grid=(NBLK,),
    in_specs=[
        pl.BlockSpec((BLK, HID // 2), lambda i: (i, 0)),
        pl.BlockSpec((BLK, HID // 2), lambda i: (i, 0)),
        pl.BlockSpec((1, 1, BLK), lambda i: (i, 0, 0)),
    ],
    out_specs=pl.BlockSpec((N_SUB, HID), lambda i: (0, 0)),
    out_shape=jax.ShapeDtypeStruct((N_SUB, HID), jnp.float32),
    scratch_shapes=[pltpu.VMEM((N_SUB, HID), jnp.float32)],
)


def _mlp2ph_body(h0, h1, a0, a1, sc, w1, b1, w2, b2, bat, p0, p1, s2g,
                 l1w, l1b, l2w, l2b, out, acc):
    i = pl.program_id(0)
    z = jnp.concatenate(
        [sc[0, 0] * h0[...] + a0[...], sc[0, 0] * h1[...] + a1[...]], axis=1)
    t = jnp.maximum(
        jnp.dot(z.astype(jnp.bfloat16), w1[...],
                preferred_element_type=jnp.float32) + b1[...], 0.0)
    y = jnp.dot(t.astype(jnp.bfloat16), w2[...],
                preferred_element_type=jnp.float32) + b2[...]
    m = (lax.broadcasted_iota(jnp.int32, (N_SUB, BLK), 0)
         == bat[0]).astype(jnp.bfloat16)
    part = jnp.dot(m, y.astype(jnp.bfloat16),
                   preferred_element_type=jnp.float32)

    @pl.when(i == 0)
    def _():
        acc[...] = part

    @pl.when(i > 0)
    def _():
        acc[...] += part

    @pl.when(i == NBLK - 1)
    def _():
        sub = jnp.concatenate([p0[...], p1[...], acc[...]], axis=1)
        mg = (lax.broadcasted_iota(jnp.int32, (N_GRAPH, N_SUB), 0)
              == s2g[0, 0]).astype(jnp.float32)
        g = jnp.dot(mg, sub, preferred_element_type=jnp.float32)
        tt = jnp.maximum(
            jnp.dot(g, l1w[...], preferred_element_type=jnp.float32)
            + l1b[...], 0.0)
        o = jnp.dot(tt, l2w[...], preferred_element_type=jnp.float32) + l2b[...]
        mx = jnp.max(o, axis=1, keepdims=True)
        lse = jnp.log(jnp.sum(jnp.exp(o - mx), axis=1, keepdims=True)) + mx
        out[...] = o - lse


_mlp2ph = pl.pallas_call(
    _mlp2ph_body,
    grid=(NBLK,),
    in_specs=[
        pl.BlockSpec((BLK, HID // 2), lambda i: (i, 0)),
        pl.BlockSpec((BLK, HID // 2), lambda i: (i, 0)),
        pl.BlockSpec((BLK, HID // 2), lambda i: (i, 0)),
        pl.BlockSpec((BLK, HID // 2), lambda i: (i, 0)),
        pl.BlockSpec((1, 1), lambda i: (0, 0)),
        pl.BlockSpec((HID, HID), lambda i: (0, 0)),
        pl.BlockSpec((HID,), lambda i: (0,)),
        pl.BlockSpec((HID, HID), lambda i: (0, 0)),
        pl.BlockSpec((HID,), lambda i: (0,)),
        pl.BlockSpec((1, 1, BLK), lambda i: (i, 0, 0)),
        pl.BlockSpec((N_SUB, HID), lambda i: (0, 0)),
        pl.BlockSpec((N_SUB, HID), lambda i: (0, 0)),
        pl.BlockSpec((1, 1, N_SUB), lambda i: (0, 0, 0)),
        pl.BlockSpec((3 * HID, HID), lambda i: (0, 0)),
        pl.BlockSpec((HID,), lambda i: (0,)),
        pl.BlockSpec((HID, HID), lambda i: (0, 0)),
        pl.BlockSpec((HID,), lambda i: (0,)),
    ],
    out_specs=pl.BlockSpec((N_GRAPH, HID), lambda i: (0, 0)),
    out_shape=jax.ShapeDtypeStruct((N_GRAPH, HID), jnp.float32),
    scratch_shapes=[pltpu.VMEM((N_SUB, HID), jnp.float32)],
)


def kernel(x, edge_index, batch, subgraph_to_graph,
           W1_0, b1_0, W2_0, b2_0, eps_0,
           W1_1, b1_1, W2_1, b2_1, eps_1,
           W1_2, b1_2, W2_2, b2_2, eps_2,
           lin1_W, lin1_b, lin2_W, lin2_b):
    src = edge_index[0]
    dst = edge_index[1]
    zer = jnp.zeros((N_NODES, DH), jnp.float32)
    bat3 = batch.astype(jnp.int32).reshape(NBLK, 1, BLK)
    s2g3 = subgraph_to_graph.astype(jnp.int32).reshape(1, 1, N_SUB)

    a0, a1 = _sc_agg_es(x, src, dst, zer)
    y00, y01 = _mlp0(x, a0, a1, (1.0 + eps_0).reshape(1, 1),
                     W1_0.astype(jnp.bfloat16), b1_0,
                     W2_0.astype(jnp.bfloat16), b2_0)
    p0 = _pool(y00, y01, bat3)

    a0, a1 = _sc_agg_fs(y00, y01, src, dst, zer)
    y10, y11 = _mlp2(y00, y01, a0, a1, (1.0 + eps_1).reshape(1, 1),
                     W1_1.astype(jnp.bfloat16), b1_1,
                     W2_1.astype(jnp.bfloat16), b2_1)
    p1 = _pool(y10, y11, bat3)

    a0, a1 = _sc_agg_fs(y10, y11, src, dst, zer)
    return _mlp2ph(y10, y11, a0, a1, (1.0 + eps_2).reshape(1, 1),
                   W1_2.astype(jnp.bfloat16), b1_2,
                   W2_2.astype(jnp.bfloat16), b2_2, bat3, p0, p1, s2g3,
                   lin1_W, lin1_b, lin2_W, lin2_b)


# speedup vs baseline: 1.0244x; 1.0056x over previous
import functools

import jax
import jax.numpy as jnp
from jax import lax
from jax.experimental import pallas as pl
from jax.experimental.pallas import tpu as pltpu
from jax.experimental.pallas import tpu_sc as plsc

N_NODES = 10000
N_EDGES = 320000
N_SUB = 500
N_GRAPH = 32
HID = 256
DH = 128

NS = 16
W = 128
ROWS_PER = 624
ROWS_TAIL = N_NODES - NS * ROWS_PER

NBLK = 10
BLK = N_NODES // NBLK


def _scratches(tail):
    return [
        pltpu.VMEM((W,), jnp.int32),
        pltpu.VMEM((W,), jnp.int32),
        pltpu.VMEM((W,), jnp.int32),
        pltpu.VMEM((W,), jnp.int32),
        pltpu.VMEM((W, DH), jnp.float32),
        pltpu.VMEM((W, DH), jnp.float32),
        pltpu.VMEM((tail,), jnp.int32),
        pltpu.VMEM((tail,), jnp.int32),
        pltpu.VMEM((tail, DH), jnp.float32),
        pltpu.VMEM_SHARED((N_NODES, DH), jnp.float32),
        pltpu.SemaphoreType.DMA,
        pltpu.SemaphoreType.DMA,
        pltpu.SemaphoreType.DMA,
        pltpu.SemaphoreType.DMA,
        pltpu.SemaphoreType.DMA,
        pltpu.SemaphoreType.DMA,
    ]


def _agg_subcore(h_ref, out_ref, src, dst, zer, aggs,
                 srcv0, srcv1, dstv0, dstv1, rows0, rows1,
                 srcv_t, dstv_t, rows_t,
                 issem0, issem1, idsem0, idsem1, gsem0, gsem1,
                 s, ebase, nwin, tail):
    pltpu.sync_copy(src.at[pl.ds(ebase, W)], srcv0)
    pltpu.async_copy(dst.at[pl.ds(ebase, W)], dstv0, idsem0)
    pltpu.async_copy(h_ref.at[srcv0], rows0, gsem0)
    pltpu.async_copy(src.at[pl.ds(ebase + W, W)], srcv1, issem1)
    pltpu.async_copy(dst.at[pl.ds(ebase + W, W)], dstv1, idsem1)

    pltpu.sync_copy(zer.at[pl.ds(s * ROWS_PER, ROWS_PER)],
                    aggs.at[pl.ds(s * ROWS_PER, ROWS_PER)])

    @pl.when(s == 0)
    def _():
        pltpu.sync_copy(zer.at[pl.ds(NS * ROWS_PER, ROWS_TAIL)],
                        aggs.at[pl.ds(NS * ROWS_PER, ROWS_TAIL)])

    plsc.subcore_barrier()

    def g_body(g, carry):
        o0 = ebase + (2 * g) * W
        o1 = o0 + W
        o2 = o1 + W
        o3 = o2 + W
        nl = g < nwin // 2 - 1

        pltpu.make_async_copy(src.at[pl.ds(o1, W)], srcv1, issem1).wait()
        pltpu.async_copy(h_ref.at[srcv1], rows1, gsem1)
        pltpu.make_async_copy(h_ref.at[srcv0], rows0, gsem0).wait()

        @pl.when(nl)
        def _():
            pltpu.async_copy(src.at[pl.ds(o2, W)], srcv0, issem0)

        pltpu.make_async_copy(dst.at[pl.ds(o0, W)], dstv0, idsem0).wait()
        pltpu.sync_copy(rows0, aggs.at[dstv0], add=True)

        @pl.when(nl)
        def _():
            pltpu.async_copy(dst.at[pl.ds(o2, W)], dstv0, idsem0)

        pltpu.make_async_copy(h_ref.at[srcv1], rows1, gsem1).wait()

        @pl.when(nl)
        def _():
            pltpu.async_copy(src.at[pl.ds(o3, W)], srcv1, issem1)

        pltpu.make_async_copy(dst.at[pl.ds(o1, W)], dstv1, idsem1).wait()
        pltpu.sync_copy(rows1, aggs.at[dstv1], add=True)

        @pl.when(nl)
        def _():
            pltpu.async_copy(dst.at[pl.ds(o3, W)], dstv1, idsem1)
            pltpu.make_async_copy(src.at[pl.ds(o2, W)], srcv0, issem0).wait()
            pltpu.async_copy(h_ref.at[srcv0], rows0, gsem0)

        return carry

    lax.fori_loop(0, nwin // 2, g_body, 0)

    off = ebase + nwin * W
    pltpu.sync_copy(src.at[pl.ds(off, tail)], srcv_t)
    pltpu.sync_copy(dst.at[pl.ds(off, tail)], dstv_t)
    pltpu.async_copy(h_ref.at[srcv_t], rows_t, gsem0).wait()
    pltpu.sync_copy(rows_t, aggs.at[dstv_t], add=True)
    plsc.subcore_barrier()
    pltpu.sync_copy(aggs.at[pl.ds(s * ROWS_PER, ROWS_PER)],
                    out_ref.at[pl.ds(s * ROWS_PER, ROWS_PER)])

    @pl.when(s == 0)
    def _():
        pltpu.sync_copy(aggs.at[pl.ds(NS * ROWS_PER, ROWS_TAIL)],
                        out_ref.at[pl.ds(NS * ROWS_PER, ROWS_TAIL)])


_MESH = plsc.VectorSubcoreMesh(core_axis_name="c", subcore_axis_name="s")
_OUT2 = (jax.ShapeDtypeStruct((N_NODES, DH), jnp.float32),
         jax.ShapeDtypeStruct((N_NODES, DH), jnp.float32))

_E_PER_FS = N_EDGES // NS
_NWIN_FS = _E_PER_FS // W
_TAIL_FS = _E_PER_FS - _NWIN_FS * W


@functools.partial(pl.kernel, out_type=_OUT2, mesh=_MESH,
                   scratch_types=_scratches(_TAIL_FS))
def _sc_agg_fs(h0, h1, src, dst, zer, out0, out1,
               srcv0, srcv1, dstv0, dstv1, rows0, rows1,
               srcv_t, dstv_t, rows_t, aggs,
               issem0, issem1, idsem0, idsem1, gsem0, gsem1):
    c = lax.axis_index("c")
    s = lax.axis_index("s")
    ebase = s * _E_PER_FS

    @pl.when(c == 0)
    def _():
        _agg_subcore(h0, out0, src, dst, zer, aggs,
                     srcv0, srcv1, dstv0, dstv1, rows0, rows1,
                     srcv_t, dstv_t, rows_t,
                     issem0, issem1, idsem0, idsem1, gsem0, gsem1,
                     s, ebase, _NWIN_FS, _TAIL_FS)

    @pl.when(c == 1)
    def _():
        _agg_subcore(h1, out1, src, dst, zer, aggs,
                     srcv0, srcv1, dstv0, dstv1, rows0, rows1,
                     srcv_t, dstv_t, rows_t,
                     issem0, issem1, idsem0, idsem1, gsem0, gsem1,
                     s, ebase, _NWIN_FS, _TAIL_FS)


_E_PER_ES = N_EDGES // (2 * NS)
_NWIN_ES = _E_PER_ES // W
_TAIL_ES = _E_PER_ES - _NWIN_ES * W


@functools.partial(pl.kernel, out_type=_OUT2, mesh=_MESH,
                   scratch_types=_scratches(_TAIL_ES))
def _sc_agg_es(h, src, dst, zer, out0, out1,
               srcv0, srcv1, dstv0, dstv1, rows0, rows1,
               srcv_t, dstv_t, rows_t, aggs,
               issem0, issem1, idsem0, idsem1, gsem0, gsem1):
    c = lax.axis_index("c")
    s = lax.axis_index("s")
    ebase = (c * NS + s) * _E_PER_ES

    @pl.when(c == 0)
    def _():
        _agg_subcore(h, out0, src, dst, zer, aggs,
                     srcv0, srcv1, dstv0, dstv1, rows0, rows1,
                     srcv_t, dstv_t, rows_t,
                     issem0, issem1, idsem0, idsem1, gsem0, gsem1,
                     s, ebase, _NWIN_ES, _TAIL_ES)

    @pl.when(c == 1)
    def _():
        _agg_subcore(h, out1, src, dst, zer, aggs,
                     srcv0, srcv1, dstv0, dstv1, rows0, rows1,
                     srcv_t, dstv_t, rows_t,
                     issem0, issem1, idsem0, idsem1, gsem0, gsem1,
                     s, ebase, _NWIN_ES, _TAIL_ES)


def _mlp0_body(h, a0, a1, sc, w1, b1, w2, b2, y0, y1):
    z = sc[0, 0] * h[...] + a0[...] + a1[...]
    t = jnp.maximum(
        jnp.dot(z.astype(jnp.bfloat16), w1[...],
                preferred_element_type=jnp.float32) + b1[...], 0.0)
    y = jnp.dot(t.astype(jnp.bfloat16), w2[...],
                preferred_element_type=jnp.float32) + b2[...]
    y0[...] = y[:, :HID // 2]
    y1[...] = y[:, HID // 2:]


_mlp0 = pl.pallas_call(
    _mlp0_body,
    grid=(NBLK,),
    in_specs=[
        pl.BlockSpec((BLK, DH), lambda i: (i, 0)),
        pl.BlockSpec((BLK, DH), lambda i: (i, 0)),
        pl.BlockSpec((BLK, DH), lambda i: (i, 0)),
        pl.BlockSpec((1, 1), lambda i: (0, 0)),
        pl.BlockSpec((DH, HID), lambda i: (0, 0)),
        pl.BlockSpec((HID,), lambda i: (0,)),
        pl.BlockSpec((HID, HID), lambda i: (0, 0)),
        pl.BlockSpec((HID,), lambda i: (0,)),
    ],
    out_specs=[
        pl.BlockSpec((BLK, HID // 2), lambda i: (i, 0)),
        pl.BlockSpec((BLK, HID // 2), lambda i: (i, 0)),
    ],
    out_shape=[
        jax.ShapeDtypeStruct((N_NODES, HID // 2), jnp.float32),
        jax.ShapeDtypeStruct((N_NODES, HID // 2), jnp.float32),
    ],
)


def _mlp2_body(h0, h1, a0, a1, sc, w1, b1, w2, b2, y0, y1):
    z = jnp.concatenate(
        [sc[0, 0] * h0[...] + a0[...], sc[0, 0] * h1[...] + a1[...]], axis=1)
    t = jnp.maximum(
        jnp.dot(z.astype(jnp.bfloat16), w1[...],
                preferred_element_type=jnp.float32) + b1[...], 0.0)
    y = jnp.dot(t.astype(jnp.bfloat16), w2[...],
                preferred_element_type=jnp.float32) + b2[...]
    y0[...] = y[:, :HID // 2]
    y1[...] = y[:, HID // 2:]


_mlp2 = pl.pallas_call(
    _mlp2_body,
    grid=(NBLK,),
    in_specs=[
        pl.BlockSpec((BLK, HID // 2), lambda i: (i, 0)),
        pl.BlockSpec((BLK, HID // 2), lambda i: (i, 0)),
        pl.BlockSpec((BLK, HID // 2), lambda i: (i, 0)),
        pl.BlockSpec((BLK, HID // 2), lambda i: (i, 0)),
        pl.BlockSpec((1, 1), lambda i: (0, 0)),
        pl.BlockSpec((HID, HID), lambda i: (0, 0)),
        pl.BlockSpec((HID,), lambda i: (0,)),
        pl.BlockSpec((HID, HID), lambda i: (0, 0)),
        pl.BlockSpec((HID,), lambda i: (0,)),
    ],
    out_specs=[
        pl.BlockSpec((BLK, HID // 2), lambda i: (i, 0)),
        pl.BlockSpec((BLK, HID // 2), lambda i: (i, 0)),
    ],
    out_shape=[
        jax.ShapeDtypeStruct((N_NODES, HID // 2), jnp.float32),
        jax.ShapeDtypeStruct((N_NODES, HID // 2), jnp.float32),
    ],
)


def _pool_body(y0, y1, bat, pool, acc):
    i = pl.program_id(0)
    xb = jnp.concatenate([y0[...], y1[...]], axis=1).astype(jnp.bfloat16)
    m = (lax.broadcasted_iota(jnp.int32, (N_SUB, BLK), 0)
         == bat[0]).astype(jnp.bfloat16)
    part = jnp.dot(m, xb, preferred_element_type=jnp.float32)

    @pl.when(i == 0)
    def _():
        acc[...] = part

    @pl.when(i > 0)
    def _():
        acc[...] += part

    @pl.when(i == NBLK - 1)
    def _():
        pool[...] = acc[...]


_pool = pl.pallas_call(
    _pool_body,
    grid=(NBLK,),
    in_specs=[
        pl.BlockSpec((BLK, HID // 2), lambda i: (i, 0)),
        pl.BlockSpec((BLK, HID // 2), lambda i: (i, 0)),
        pl.BlockSpec((1, 1, BLK), lambda i: (i, 0, 0)),
    ],
    out_specs=pl.BlockSpec((N_SUB, HID), lambda i: (0, 0)),
    out_shape=jax.ShapeDtypeStruct((N_SUB, HID), jnp.float32),
    scratch_shapes=[pltpu.VMEM((N_SUB, HID), jnp.float32)],
)


def _mlp2ph_body(h0, h1, a0, a1, sc, w1, b1, w2, b2, bat, p0, p1, s2g,
                 l1w, l1b, l2w, l2b, out, acc):
    i = pl.program_id(0)
    z = jnp.concatenate(
        [sc[0, 0] * h0[...] + a0[...], sc[0, 0] * h1[...] + a1[...]], axis=1)
    t = jnp.maximum(
        jnp.dot(z.astype(jnp.bfloat16), w1[...],
                preferred_element_type=jnp.float32) + b1[...], 0.0)
    y = jnp.dot(t.astype(jnp.bfloat16), w2[...],
                preferred_element_type=jnp.float32) + b2[...]
    m = (lax.broadcasted_iota(jnp.int32, (N_SUB, BLK), 0)
         == bat[0]).astype(jnp.bfloat16)
    part = jnp.dot(m, y.astype(jnp.bfloat16),
                   preferred_element_type=jnp.float32)

    @pl.when(i == 0)
    def _():
        acc[...] = part

    @pl.when(i > 0)
    def _():
        acc[...] += part

    @pl.when(i == NBLK - 1)
    def _():
        sub = jnp.concatenate([p0[...], p1[...], acc[...]], axis=1)
        mg = (lax.broadcasted_iota(jnp.int32, (N_GRAPH, N_SUB), 0)
              == s2g[0, 0]).astype(jnp.float32)
        g = jnp.dot(mg, sub, preferred_element_type=jnp.float32)
        tt = jnp.maximum(
            jnp.dot(g, l1w[...], preferred_element_type=jnp.float32)
            + l1b[...], 0.0)
        o = jnp.dot(tt, l2w[...], preferred_element_type=jnp.float32) + l2b[...]
        mx = jnp.max(o, axis=1, keepdims=True)
        lse = jnp.log(jnp.sum(jnp.exp(o - mx), axis=1, keepdims=True)) + mx
        out[...] = o - lse


_mlp2ph = pl.pallas_call(
    _mlp2ph_body,
    grid=(NBLK,),
    in_specs=[
        pl.BlockSpec((BLK, HID // 2), lambda i: (i, 0)),
        pl.BlockSpec((BLK, HID // 2), lambda i: (i, 0)),
        pl.BlockSpec((BLK, HID // 2), lambda i: (i, 0)),
        pl.BlockSpec((BLK, HID // 2), lambda i: (i, 0)),
        pl.BlockSpec((1, 1), lambda i: (0, 0)),
        pl.BlockSpec((HID, HID), lambda i: (0, 0)),
        pl.BlockSpec((HID,), lambda i: (0,)),
        pl.BlockSpec((HID, HID), lambda i: (0, 0)),
        pl.BlockSpec((HID,), lambda i: (0,)),
        pl.BlockSpec((1, 1, BLK), lambda i: (i, 0, 0)),
        pl.BlockSpec((N_SUB, HID), lambda i: (0, 0)),
        pl.BlockSpec((N_SUB, HID), lambda i: (0, 0)),
        pl.BlockSpec((1, 1, N_SUB), lambda i: (0, 0, 0)),
        pl.BlockSpec((3 * HID, HID), lambda i: (0, 0)),
        pl.BlockSpec((HID,), lambda i: (0,)),
        pl.BlockSpec((HID, HID), lambda i: (0, 0)),
        pl.BlockSpec((HID,), lambda i: (0,)),
    ],
    out_specs=pl.BlockSpec((N_GRAPH, HID), lambda i: (0, 0)),
    out_shape=jax.ShapeDtypeStruct((N_GRAPH, HID), jnp.float32),
    scratch_shapes=[pltpu.VMEM((N_SUB, HID), jnp.float32)],
)


def kernel(x, edge_index, batch, subgraph_to_graph,
           W1_0, b1_0, W2_0, b2_0, eps_0,
           W1_1, b1_1, W2_1, b2_1, eps_1,
           W1_2, b1_2, W2_2, b2_2, eps_2,
           lin1_W, lin1_b, lin2_W, lin2_b):
    src = edge_index[0]
    dst = edge_index[1]
    zer = jnp.zeros((N_NODES, DH), jnp.float32)
    bat3 = batch.astype(jnp.int32).reshape(NBLK, 1, BLK)
    s2g3 = subgraph_to_graph.astype(jnp.int32).reshape(1, 1, N_SUB)

    a0, a1 = _sc_agg_es(x, src, dst, zer)
    y00, y01 = _mlp0(x, a0, a1, (1.0 + eps_0).reshape(1, 1),
                     W1_0.astype(jnp.bfloat16), b1_0,
                     W2_0.astype(jnp.bfloat16), b2_0)
    p0 = _pool(y00, y01, bat3)

    a0, a1 = _sc_agg_fs(y00, y01, src, dst, zer)
    y10, y11 = _mlp2(y00, y01, a0, a1, (1.0 + eps_1).reshape(1, 1),
                     W1_1.astype(jnp.bfloat16), b1_1,
                     W2_1.astype(jnp.bfloat16), b2_1)
    p1 = _pool(y10, y11, bat3)

    a0, a1 = _sc_agg_fs(y10, y11, src, dst, zer)
    return _mlp2ph(y10, y11, a0, a1, (1.0 + eps_2).reshape(1, 1),
                   W1_2.astype(jnp.bfloat16), b1_2,
                   W2_2.astype(jnp.bfloat16), b2_2, bat3, p0, p1, s2g3,
                   lin1_W, lin1_b, lin2_W, lin2_b)
